# Initial kernel scaffold; baseline (speedup 1.0000x reference)
import jax
import jax.numpy as jnp
from jax.experimental import pallas as pl


def kernel(x, edge_index, params):
    raise NotImplementedError("write your pallas kernel here")


# speedup vs baseline: 8.6996x; 8.6996x over previous
import functools

import jax
import jax.numpy as jnp
from jax import lax
from jax.experimental import pallas as pl
from jax.experimental.pallas import tpu as pltpu
from jax.experimental.pallas import tpu_sc as plsc

_N = 10000
_E = 320000
_DIMS = [128] + [256] * 7 + [128] * 7 + [64] * 7 + [32] * 7 + [16] * 7 + [8] * 7 + [4] * 6

_NP = 10240
_NC, _NS = 2, 16
_CH = 125
_EPT = _E // (_NC * _NS)
_NIT = _EPT // _CH
_RPT = _NP // _NS
_ZR = 32

_MB = 2000


def _make_spmm(F):
    mesh = plsc.VectorSubcoreMesh(
        core_axis_name="c", subcore_axis_name="s", num_cores=_NC, num_subcores=_NS
    )

    @functools.partial(
        pl.kernel,
        out_type=jax.ShapeDtypeStruct((2, _NP, F), jnp.float32),
        mesh=mesh,
        scratch_types=[
            pltpu.VMEM((_NIT, _CH), jnp.int32),
            pltpu.VMEM((_NIT, _CH), jnp.int32),
            pltpu.VMEM((_CH, F), jnp.float32),
            pltpu.VMEM((_ZR, F), jnp.float32),
            pltpu.VMEM_SHARED((_NP, F), jnp.float32),
            pltpu.SemaphoreType.DMA,
        ],
        compiler_params=pltpu.CompilerParams(use_tc_tiling_on_sc=False),
        name=f"sage_spmm_f{F}",
    )
    def spmm(z_hbm, src_hbm, dst_hbm, out_hbm, src_v, dst_v, rows_v, zero_v, acc_sh, sem):
        c = lax.axis_index("c")
        s = lax.axis_index("s")
        wid = s * _NC + c
        for a in range(_ZR):
            for k in range(F // 16):
                zero_v[a, pl.ds(k * 16, 16)] = jnp.zeros((16,), jnp.float32)
        row0 = s * _RPT
        for a in range(_RPT // _ZR):
            pltpu.sync_copy(zero_v, acc_sh.at[pl.ds(row0 + a * _ZR, _ZR)])
        plsc.subcore_barrier()
        pltpu.sync_copy(src_hbm.at[pl.ds(wid * _NIT, _NIT)], src_v)
        pltpu.sync_copy(dst_hbm.at[pl.ds(wid * _NIT, _NIT)], dst_v)

        def body(j, carry):
            pltpu.async_copy(z_hbm.at[src_v.at[j]], rows_v, sem).wait()
            pltpu.sync_copy(rows_v, acc_sh.at[dst_v.at[j]], add=True)
            return carry

        lax.fori_loop(0, _NIT, body, 0)
        plsc.subcore_barrier()
        pltpu.sync_copy(
            acc_sh.at[pl.ds(row0, _RPT)], out_hbm.at[c, pl.ds(row0, _RPT)]
        )

    return spmm


_SPMM_CACHE = {}


def _spmm(z, src2, dst2):
    F = z.shape[1]
    if F not in _SPMM_CACHE:
        _SPMM_CACHE[F] = _make_spmm(F)
    return _SPMM_CACHE[F](z, src2, dst2)


def _dinv_block(degp_blk):
    deg = degp_blk[0, :, 0:1] + degp_blk[1, :, 0:1]
    return jnp.where(deg > 0.0, 1.0 / jnp.maximum(deg, 1.0), 0.0)


def _row_spec(width):
    return pl.BlockSpec((_MB, width), lambda m: (m, 0))


def _part_spec(width):
    return pl.BlockSpec((2, _MB, width), lambda m: (0, m, 0))


def _full_spec(*shape):
    return pl.BlockSpec(shape, lambda m: (0,) * len(shape))


def _h_store_shapes(fo):
    if fo == 256:
        return [128, 128]
    return [max(16, fo)]


def _split_h(h, fo):
    if fo == 256:
        return [h[:, :128], h[:, 128:]]
    if fo < 16:
        h = jnp.concatenate([h, jnp.zeros((h.shape[0], 16 - fo), jnp.float32)], axis=1)
    return [h]


def _make_tc_layer(fi, fo, last):
    sp_ws = _h_store_shapes(fi)
    h_ws = _h_store_shapes(fi)
    nsp, nh = len(sp_ws), len(h_ws)
    if last:
        out_ws = [1]
    else:
        out_ws = _h_store_shapes(fo)

    def body(*refs):
        sp_refs = refs[:nsp]
        h_refs = refs[nsp:nsp + nh]
        degp_ref, b_ref, wl_ref, wr_ref = refs[nsp + nh:nsp + nh + 4]
        pos = nsp + nh + 4
        if last:
            wreg_ref, breg_ref = refs[pos:pos + 2]
            pos += 2
        out_refs = refs[pos:]

        dinv = _dinv_block(degp_ref[...])
        s = jnp.concatenate([r[0] + r[1] for r in sp_refs], axis=1)[:, :fi]
        h = jnp.concatenate([r[...] for r in h_refs], axis=1)[:, :fi]
        agg = s * dinv
        hn = jnp.maximum(
            jnp.dot(agg, wl_ref[...], preferred_element_type=jnp.float32)
            + jnp.dot(h, wr_ref[...], preferred_element_type=jnp.float32)
            + b_ref[...],
            0.0,
        )
        if last:
            out_refs[0][...] = (
                jnp.dot(hn, wreg_ref[...], preferred_element_type=jnp.float32)
                + breg_ref[...]
            )
        else:
            for o_ref, part in zip(out_refs, _split_h(hn, fo)):
                o_ref[...] = part

    in_specs = (
        [_part_spec(w) for w in sp_ws]
        + [_row_spec(w) for w in h_ws]
        + [_part_spec(16), _full_spec(1, fo), _full_spec(fi, fo), _full_spec(fi, fo)]
    )
    if last:
        in_specs += [_full_spec(fo, 1), _full_spec(1, 1)]
    out_specs = [_row_spec(w) for w in out_ws]
    out_shape = [jax.ShapeDtypeStruct((_N, w), jnp.float32) for w in out_ws]

    def run(sps, hs, degp, b, Wl, Wr, head=None):
        args = list(sps) + list(hs) + [degp, b, Wl, Wr]
        if last:
            args += [head[0], head[1]]
        outs = pl.pallas_call(
            body,
            grid=(_N // _MB,),
            in_specs=in_specs,
            out_specs=out_specs,
            out_shape=out_shape,
        )(*args)
        return outs

    return run


def kernel(x, edge_index, params):
    src2 = edge_index[0].reshape(_E // _CH, _CH)
    dst2 = edge_index[1].reshape(_E // _CH, _CH)

    ones16 = jnp.ones((_N, 16), jnp.float32)
    degp = _spmm(ones16, src2, dst2)

    hs = [x]
    for i in range(48):
        fi, fo = _DIMS[i], _DIMS[i + 1]
        last = i == 47
        sps = [_spmm(h, src2, dst2) for h in hs]
        tc = _make_tc_layer(fi, fo, last)
        outs = tc(
            sps, hs, degp, params[f"b_{i}"].reshape(1, -1),
            params[f"Wl_{i}"], params[f"Wr_{i}"],
            head=(params["W_reg"], params["b_reg"].reshape(1, 1)) if last else None,
        )
        hs = list(outs)
    return hs[0]


# speedup vs baseline: 9.1615x; 1.0531x over previous
import functools

import jax
import jax.numpy as jnp
from jax import lax
from jax.experimental import pallas as pl
from jax.experimental.pallas import tpu as pltpu
from jax.experimental.pallas import tpu_sc as plsc

_N = 10000
_E = 320000
_DIMS = [128] + [256] * 7 + [128] * 7 + [64] * 7 + [32] * 7 + [16] * 7 + [8] * 7 + [4] * 6

_NP = 10240
_NC, _NS = 2, 16
_CH = 50
_EPT = _E // (_NC * _NS)
_NIT = _EPT // _CH
_RPT = _NP // _NS
_ZR = 32
_UNR = 10

_MB = 2000


def _make_spmm(F):
    mesh = plsc.VectorSubcoreMesh(
        core_axis_name="c", subcore_axis_name="s", num_cores=_NC, num_subcores=_NS
    )

    @functools.partial(
        pl.kernel,
        out_type=jax.ShapeDtypeStruct((2, _NP, F), jnp.float32),
        mesh=mesh,
        scratch_types=[
            pltpu.VMEM((_NIT, _CH), jnp.int32),
            pltpu.VMEM((_NIT, _CH), jnp.int32),
            pltpu.VMEM((_CH, F), jnp.float32),
            pltpu.VMEM((_CH, F), jnp.float32),
            pltpu.VMEM((_ZR, F), jnp.float32),
            pltpu.VMEM_SHARED((_NP, F), jnp.float32),
            pltpu.SemaphoreType.DMA,
            pltpu.SemaphoreType.DMA,
        ],
        compiler_params=pltpu.CompilerParams(use_tc_tiling_on_sc=False),
        name=f"sage_spmm_f{F}",
    )
    def spmm(z_hbm, src_hbm, dst_hbm, out_hbm, src_v, dst_v, rows0_v, rows1_v,
             zero_v, acc_sh, sem0, sem1):
        c = lax.axis_index("c")
        s = lax.axis_index("s")
        wid = s * _NC + c
        for a in range(_ZR):
            for k in range(F // 16):
                zero_v[a, pl.ds(k * 16, 16)] = jnp.zeros((16,), jnp.float32)
        row0 = s * _RPT
        for a in range(_RPT // _ZR):
            pltpu.sync_copy(zero_v, acc_sh.at[pl.ds(row0 + a * _ZR, _ZR)])
        plsc.subcore_barrier()
        pltpu.sync_copy(src_hbm.at[pl.ds(wid * _NIT, _NIT)], src_v)
        pltpu.sync_copy(dst_hbm.at[pl.ds(wid * _NIT, _NIT)], dst_v)

        bufs = (rows0_v, rows1_v)
        sems = (sem0, sem1)

        def body(o, carry):
            j0 = o * _UNR
            descs = [None] * _UNR
            descs[0] = pltpu.async_copy(z_hbm.at[src_v.at[j0]], bufs[0], sems[0])
            descs[1] = pltpu.async_copy(z_hbm.at[src_v.at[j0 + 1]], bufs[1], sems[1])
            for k in range(_UNR):
                descs[k].wait()
                pltpu.sync_copy(bufs[k % 2], acc_sh.at[dst_v.at[j0 + k]], add=True)
                if k + 2 < _UNR:
                    descs[k + 2] = pltpu.async_copy(
                        z_hbm.at[src_v.at[j0 + k + 2]], bufs[k % 2], sems[k % 2]
                    )
            return carry

        lax.fori_loop(0, _NIT // _UNR, body, 0)
        plsc.subcore_barrier()
        pltpu.sync_copy(
            acc_sh.at[pl.ds(row0, _RPT)], out_hbm.at[c, pl.ds(row0, _RPT)]
        )

    return spmm


_SPMM_CACHE = {}


def _spmm(z, src2, dst2):
    F = z.shape[1]
    if F not in _SPMM_CACHE:
        _SPMM_CACHE[F] = _make_spmm(F)
    return _SPMM_CACHE[F](z, src2, dst2)


def _dinv_block(degp_blk):
    deg = degp_blk[0, :, 0:1] + degp_blk[1, :, 0:1]
    return jnp.where(deg > 0.0, 1.0 / jnp.maximum(deg, 1.0), 0.0)


def _row_spec(width):
    return pl.BlockSpec((_MB, width), lambda m: (m, 0))


def _part_spec(width):
    return pl.BlockSpec((2, _MB, width), lambda m: (0, m, 0))


def _full_spec(*shape):
    return pl.BlockSpec(shape, lambda m: (0,) * len(shape))


def _h_store_shapes(fo):
    if fo == 256:
        return [128, 128]
    return [max(16, fo)]


def _split_h(h, fo):
    if fo == 256:
        return [h[:, :128], h[:, 128:]]
    if fo < 16:
        h = jnp.concatenate([h, jnp.zeros((h.shape[0], 16 - fo), jnp.float32)], axis=1)
    return [h]


def _make_tc_layer(fi, fo, last):
    sp_ws = _h_store_shapes(fi)
    h_ws = _h_store_shapes(fi)
    nsp, nh = len(sp_ws), len(h_ws)
    if last:
        out_ws = [1]
    else:
        out_ws = _h_store_shapes(fo)

    def body(*refs):
        sp_refs = refs[:nsp]
        h_refs = refs[nsp:nsp + nh]
        degp_ref, b_ref, wl_ref, wr_ref = refs[nsp + nh:nsp + nh + 4]
        pos = nsp + nh + 4
        if last:
            wreg_ref, breg_ref = refs[pos:pos + 2]
            pos += 2
        out_refs = refs[pos:]

        dinv = _dinv_block(degp_ref[...])
        s = jnp.concatenate([r[0] + r[1] for r in sp_refs], axis=1)[:, :fi]
        h = jnp.concatenate([r[...] for r in h_refs], axis=1)[:, :fi]
        agg = s * dinv
        hn = jnp.maximum(
            jnp.dot(agg, wl_ref[...], preferred_element_type=jnp.float32)
            + jnp.dot(h, wr_ref[...], preferred_element_type=jnp.float32)
            + b_ref[...],
            0.0,
        )
        if last:
            out_refs[0][...] = (
                jnp.dot(hn, wreg_ref[...], preferred_element_type=jnp.float32)
                + breg_ref[...]
            )
        else:
            for o_ref, part in zip(out_refs, _split_h(hn, fo)):
                o_ref[...] = part

    in_specs = (
        [_part_spec(w) for w in sp_ws]
        + [_row_spec(w) for w in h_ws]
        + [_part_spec(16), _full_spec(1, fo), _full_spec(fi, fo), _full_spec(fi, fo)]
    )
    if last:
        in_specs += [_full_spec(fo, 1), _full_spec(1, 1)]
    out_specs = [_row_spec(w) for w in out_ws]
    out_shape = [jax.ShapeDtypeStruct((_N, w), jnp.float32) for w in out_ws]

    def run(sps, hs, degp, b, Wl, Wr, head=None):
        args = list(sps) + list(hs) + [degp, b, Wl, Wr]
        if last:
            args += [head[0], head[1]]
        outs = pl.pallas_call(
            body,
            grid=(_N // _MB,),
            in_specs=in_specs,
            out_specs=out_specs,
            out_shape=out_shape,
        )(*args)
        return outs

    return run


def kernel(x, edge_index, params):
    src2 = edge_index[0].reshape(_E // _CH, _CH)
    dst2 = edge_index[1].reshape(_E // _CH, _CH)

    ones16 = jnp.ones((_N, 16), jnp.float32)
    degp = _spmm(ones16, src2, dst2)

    hs = [x]
    for i in range(48):
        fi, fo = _DIMS[i], _DIMS[i + 1]
        last = i == 47
        sps = [_spmm(h, src2, dst2) for h in hs]
        tc = _make_tc_layer(fi, fo, last)
        outs = tc(
            sps, hs, degp, params[f"b_{i}"].reshape(1, -1),
            params[f"Wl_{i}"], params[f"Wr_{i}"],
            head=(params["W_reg"], params["b_reg"].reshape(1, 1)) if last else None,
        )
        hs = list(outs)
    return hs[0]


# speedup vs baseline: 12.8399x; 1.4015x over previous
import functools

import jax
import jax.numpy as jnp
from jax import lax
from jax.experimental import pallas as pl
from jax.experimental.pallas import tpu as pltpu
from jax.experimental.pallas import tpu_sc as plsc

_N = 10000
_E = 320000
_DIMS = [128] + [256] * 7 + [128] * 7 + [64] * 7 + [32] * 7 + [16] * 7 + [8] * 7 + [4] * 6

_NP = 10240
_NC, _NS = 2, 16
_CH = 125
_EPT = _E // (_NC * _NS)
_NIT = _EPT // _CH
_RPT = _NP // _NS
_ZR = 32
_UNR = 10
_NBLK = _NIT // _UNR

_MB = 2000


def _make_spmm(F):
    mesh = plsc.VectorSubcoreMesh(
        core_axis_name="c", subcore_axis_name="s", num_cores=_NC, num_subcores=_NS
    )

    @functools.partial(
        pl.kernel,
        out_type=jax.ShapeDtypeStruct((2, _NP, F), jnp.float32),
        mesh=mesh,
        scratch_types=[
            pltpu.VMEM((2 * _UNR, 2, _CH), jnp.int32),
            pltpu.VMEM((_CH, F), jnp.float32),
            pltpu.VMEM((_CH, F), jnp.float32),
            pltpu.VMEM((_ZR, F), jnp.float32),
            pltpu.VMEM_SHARED((_NP, F), jnp.float32),
            pltpu.SemaphoreType.DMA,
            pltpu.SemaphoreType.DMA,
            pltpu.SemaphoreType.DMA,
        ],
        compiler_params=pltpu.CompilerParams(use_tc_tiling_on_sc=False),
        name=f"sage_spmm_f{F}",
    )
    def spmm(z_hbm, ei_hbm, out_hbm, idx_v, rows0_v, rows1_v,
             zero_v, acc_sh, sem0, sem1, semi):
        c = lax.axis_index("c")
        s = lax.axis_index("s")
        wid = s * _NC + c
        for a in range(_ZR):
            for k in range(F // 16):
                zero_v[a, pl.ds(k * 16, 16)] = jnp.zeros((16,), jnp.float32)
        row0 = s * _RPT
        for a in range(_RPT // _ZR):
            pltpu.sync_copy(zero_v, acc_sh.at[pl.ds(row0 + a * _ZR, _ZR)])
        plsc.subcore_barrier()

        base = wid * _NIT
        pltpu.async_copy(
            ei_hbm.at[pl.ds(base, _UNR)], idx_v.at[pl.ds(0, _UNR)], semi
        )
        bufs = (rows0_v, rows1_v)
        sems = (sem0, sem1)

        def body(o, carry):
            p = lax.rem(o, 2)
            q = 1 - p
            ioff = p * _UNR
            pltpu.make_async_copy(
                ei_hbm.at[pl.ds(base, _UNR)], idx_v.at[pl.ds(0, _UNR)], semi
            ).wait()

            @pl.when(o + 1 < _NBLK)
            def _():
                pltpu.async_copy(
                    ei_hbm.at[pl.ds(base + (o + 1) * _UNR, _UNR)],
                    idx_v.at[pl.ds(q * _UNR, _UNR)],
                    semi,
                )

            descs = [None] * _UNR
            descs[0] = pltpu.async_copy(z_hbm.at[idx_v.at[ioff, 0]], bufs[0], sems[0])
            descs[1] = pltpu.async_copy(
                z_hbm.at[idx_v.at[ioff + 1, 0]], bufs[1], sems[1]
            )
            for k in range(_UNR):
                descs[k].wait()
                pltpu.sync_copy(
                    bufs[k % 2], acc_sh.at[idx_v.at[ioff + k, 1]], add=True
                )
                if k + 2 < _UNR:
                    descs[k + 2] = pltpu.async_copy(
                        z_hbm.at[idx_v.at[ioff + k + 2, 0]], bufs[k % 2], sems[k % 2]
                    )
            return carry

        lax.fori_loop(0, _NBLK, body, 0)
        plsc.subcore_barrier()
        pltpu.sync_copy(
            acc_sh.at[pl.ds(row0, _RPT)], out_hbm.at[c, pl.ds(row0, _RPT)]
        )

    return spmm


_SPMM_CACHE = {}


def _spmm(z, ei3):
    F = z.shape[1]
    if F not in _SPMM_CACHE:
        _SPMM_CACHE[F] = _make_spmm(F)
    return _SPMM_CACHE[F](z, ei3)


def _dinv_block(degp_blk):
    deg = degp_blk[0, :, 0:1] + degp_blk[1, :, 0:1]
    return jnp.where(deg > 0.0, 1.0 / jnp.maximum(deg, 1.0), 0.0)


def _row_spec(width):
    return pl.BlockSpec((_MB, width), lambda m: (m, 0))


def _part_spec(width):
    return pl.BlockSpec((2, _MB, width), lambda m: (0, m, 0))


def _full_spec(*shape):
    return pl.BlockSpec(shape, lambda m: (0,) * len(shape))


def _h_store_shapes(fo):
    if fo == 256:
        return [128, 128]
    return [max(16, fo)]


def _split_h(h, fo):
    if fo == 256:
        return [h[:, :128], h[:, 128:]]
    if fo < 16:
        h = jnp.concatenate([h, jnp.zeros((h.shape[0], 16 - fo), jnp.float32)], axis=1)
    return [h]


def _make_tc_layer(fi, fo, last):
    sp_ws = _h_store_shapes(fi)
    h_ws = _h_store_shapes(fi)
    nsp, nh = len(sp_ws), len(h_ws)
    if last:
        out_ws = [1]
    else:
        out_ws = _h_store_shapes(fo)

    def body(*refs):
        sp_refs = refs[:nsp]
        h_refs = refs[nsp:nsp + nh]
        degp_ref, b_ref, wl_ref, wr_ref = refs[nsp + nh:nsp + nh + 4]
        pos = nsp + nh + 4
        if last:
            wreg_ref, breg_ref = refs[pos:pos + 2]
            pos += 2
        out_refs = refs[pos:]

        dinv = _dinv_block(degp_ref[...])
        s = jnp.concatenate([r[0] + r[1] for r in sp_refs], axis=1)[:, :fi]
        h = jnp.concatenate([r[...] for r in h_refs], axis=1)[:, :fi]
        agg = s * dinv
        hn = jnp.maximum(
            jnp.dot(agg, wl_ref[...], preferred_element_type=jnp.float32)
            + jnp.dot(h, wr_ref[...], preferred_element_type=jnp.float32)
            + b_ref[...],
            0.0,
        )
        if last:
            out_refs[0][...] = (
                jnp.dot(hn, wreg_ref[...], preferred_element_type=jnp.float32)
                + breg_ref[...]
            )
        else:
            for o_ref, part in zip(out_refs, _split_h(hn, fo)):
                o_ref[...] = part

    in_specs = (
        [_part_spec(w) for w in sp_ws]
        + [_row_spec(w) for w in h_ws]
        + [_part_spec(16), _full_spec(1, fo), _full_spec(fi, fo), _full_spec(fi, fo)]
    )
    if last:
        in_specs += [_full_spec(fo, 1), _full_spec(1, 1)]
    out_specs = [_row_spec(w) for w in out_ws]
    out_shape = [jax.ShapeDtypeStruct((_N, w), jnp.float32) for w in out_ws]

    def run(sps, hs, degp, b, Wl, Wr, head=None):
        args = list(sps) + list(hs) + [degp, b, Wl, Wr]
        if last:
            args += [head[0], head[1]]
        outs = pl.pallas_call(
            body,
            grid=(_N // _MB,),
            in_specs=in_specs,
            out_specs=out_specs,
            out_shape=out_shape,
        )(*args)
        return outs

    return run


def kernel(x, edge_index, params):
    ei3 = jnp.stack(
        [edge_index[0].reshape(_E // _CH, _CH), edge_index[1].reshape(_E // _CH, _CH)],
        axis=1,
    )

    ones16 = jnp.ones((_N, 16), jnp.float32)
    degp = _spmm(ones16, ei3)

    hs = [x]
    for i in range(48):
        fi, fo = _DIMS[i], _DIMS[i + 1]
        last = i == 47
        sps = [_spmm(h, ei3) for h in hs]
        tc = _make_tc_layer(fi, fo, last)
        outs = tc(
            sps, hs, degp, params[f"b_{i}"].reshape(1, -1),
            params[f"Wl_{i}"], params[f"Wr_{i}"],
            head=(params["W_reg"], params["b_reg"].reshape(1, 1)) if last else None,
        )
        hs = list(outs)
    return hs[0]
